# baseline (speedup 1.0000x reference)
import jax
import jax.numpy as jnp
from jax import lax
from jax.experimental import pallas as pl
from jax.experimental.pallas import tpu as pltpu, tpu_sc as plsc

T = 4096
D = 128
LANES = 16
NC = 2
NS = 16
NW = NC * NS
CHUNK = T // NW
CHUNK_E = CHUNK * D
CHUNK_B = CHUNK * LANES
KD = D // LANES

_mesh = plsc.VectorSubcoreMesh(core_axis_name="c", subcore_axis_name="s")
_cparams = pltpu.CompilerParams(needs_layout_passes=False)


def _cu_scalars(cu_vec):
    lane = lax.iota(jnp.int32, LANES)
    cu_f = cu_vec.astype(jnp.float32)
    c1 = jnp.sum(jnp.where(lane == 1, cu_f, 0.0)).astype(jnp.int32)
    c2 = jnp.sum(jnp.where(lane == 2, cu_f, 0.0)).astype(jnp.int32)
    c3 = jnp.sum(jnp.where(lane == 3, cu_f, 0.0)).astype(jnp.int32)
    return c1, c2, c3


def _seg_of(p, c1, c2, c3):
    return ((p >= c1).astype(jnp.int32) + (p >= c2).astype(jnp.int32)
            + (p >= c3).astype(jnp.int32))


def _score_pass(ctx_v, th, eb_v):
    def sbody(g, _):
        base = g * (LANES * D)
        for j in range(LANES):
            off = base + j * D
            acc = ctx_v[pl.ds(off, LANES)] * th[0]
            for k in range(1, KD):
                acc = acc + ctx_v[pl.ds(off + LANES * k, LANES)] * th[k]
            e = jnp.exp(jnp.full((LANES,), jnp.sum(acc), jnp.float32))
            eb_v[pl.ds((g * LANES + j) * LANES, LANES)] = e
        return 0

    lax.fori_loop(0, CHUNK // LANES, sbody, 0)


def _cuts(row0, c1, c2, c3):
    cuts = []
    resets = []
    for c in (c1, c2, c3):
        rel = c - row0
        inside = (rel >= 0) & (rel < CHUNK)
        cuts.append(jnp.clip(rel, 0, CHUNK))
        resets.append(jnp.full((LANES,), jnp.where(inside, 0.0, 1.0),
                               jnp.float32))
    return cuts, resets


def _phase1_body(ctx_hbm, cu_hbm, th_hbm, tnum_hbm, tden_hbm, ebc_hbm,
                 ctx_v, th_v, cu_v, tn_v, td_v, eb_v):
    c = lax.axis_index("c")
    s = lax.axis_index("s")
    wid = s * NC + c
    pltpu.sync_copy(ctx_hbm.at[pl.ds(wid * CHUNK_E, CHUNK_E)], ctx_v)
    pltpu.sync_copy(th_hbm, th_v)
    pltpu.sync_copy(cu_hbm, cu_v)
    c1, c2, c3 = _cu_scalars(cu_v[:])
    th = [th_v[pl.ds(LANES * k, LANES)] for k in range(KD)]
    row0 = wid * CHUNK
    zero = jnp.zeros((LANES,), jnp.float32)

    _score_pass(ctx_v, th, eb_v)

    def tbody(r, carry):
        den = carry[0]
        nums = carry[1:]
        off = r * D
        e = eb_v[pl.ds(r * LANES, LANES)]
        den = den + e
        nums = tuple(n + e * ctx_v[pl.ds(off + LANES * k, LANES)]
                     for k, n in enumerate(nums))
        return (den,) + nums

    cuts, resets = _cuts(row0, c1, c2, c3)
    bounds = [0] + cuts + [CHUNK]
    carry = (zero,) * (KD + 1)
    for k in range(4):
        if k > 0:
            carry = tuple(v * resets[k - 1] for v in carry)
        carry = lax.fori_loop(bounds[k], bounds[k + 1], tbody, carry)

    td_v[:] = carry[0]
    for k in range(KD):
        tn_v[pl.ds(LANES * k, LANES)] = carry[1 + k]
    pltpu.sync_copy(tn_v, tnum_hbm.at[pl.ds(wid * D, D)])
    pltpu.sync_copy(td_v, tden_hbm.at[pl.ds(wid * LANES, LANES)])
    pltpu.sync_copy(eb_v, ebc_hbm.at[pl.ds(wid * CHUNK_B, CHUNK_B)])


def _phase2_body(ctx_hbm, cu_hbm, tnum_hbm, tden_hbm, ebc_hbm, out_hbm,
                 ctx_v, out_v, cu_v, tn_v, td_v, eb_v):
    c = lax.axis_index("c")
    s = lax.axis_index("s")
    wid = s * NC + c
    pltpu.sync_copy(ctx_hbm.at[pl.ds(wid * CHUNK_E, CHUNK_E)], ctx_v)
    pltpu.sync_copy(cu_hbm, cu_v)
    pltpu.sync_copy(tnum_hbm, tn_v)
    pltpu.sync_copy(tden_hbm, td_v)
    pltpu.sync_copy(ebc_hbm.at[pl.ds(wid * CHUNK_B, CHUNK_B)], eb_v)
    c1, c2, c3 = _cu_scalars(cu_v[:])
    row0 = wid * CHUNK
    zero = jnp.zeros((LANES,), jnp.float32)

    s0 = _seg_of(row0, c1, c2, c3)
    cden = zero
    cnum = [zero] * KD
    for wp in range(NW - 1):
        segl = _seg_of(wp * CHUNK + CHUNK - 1, c1, c2, c3)
        take = (wp < wid) & (segl == s0)
        mv = jnp.full((LANES,), jnp.where(take, 1.0, 0.0), jnp.float32)
        cden = cden + mv * td_v[pl.ds(wp * LANES, LANES)]
        for k in range(KD):
            cnum[k] = cnum[k] + mv * tn_v[pl.ds(wp * D + LANES * k, LANES)]

    def body(r, carry):
        den = carry[0]
        nums = carry[1:]
        off = r * D
        e = eb_v[pl.ds(r * LANES, LANES)]
        den = den + e
        nums = tuple(n + e * ctx_v[pl.ds(off + LANES * k, LANES)]
                     for k, n in enumerate(nums))
        inv = 1.0 / den
        for k in range(KD):
            out_v[pl.ds(off + LANES * k, LANES)] = nums[k] * inv
        return (den,) + nums

    cuts, resets = _cuts(row0, c1, c2, c3)
    bounds = [0] + cuts + [CHUNK]
    carry = (cden,) + tuple(cnum)
    for k in range(4):
        if k > 0:
            carry = tuple(v * resets[k - 1] for v in carry)
        carry = lax.fori_loop(bounds[k], bounds[k + 1], body, carry)

    pltpu.sync_copy(out_v, out_hbm.at[pl.ds(wid * CHUNK_E, CHUNK_E)])


_phase1 = pl.kernel(
    _phase1_body,
    out_type=(jax.ShapeDtypeStruct((NW * D,), jnp.float32),
              jax.ShapeDtypeStruct((NW * LANES,), jnp.float32),
              jax.ShapeDtypeStruct((T * LANES,), jnp.float32)),
    mesh=_mesh,
    compiler_params=_cparams,
    scratch_types=[
        pltpu.VMEM((CHUNK_E,), jnp.float32),
        pltpu.VMEM((D,), jnp.float32),
        pltpu.VMEM((LANES,), jnp.int32),
        pltpu.VMEM((D,), jnp.float32),
        pltpu.VMEM((LANES,), jnp.float32),
        pltpu.VMEM((CHUNK_B,), jnp.float32),
    ],
)

_phase2 = pl.kernel(
    _phase2_body,
    out_type=jax.ShapeDtypeStruct((T * D,), jnp.float32),
    mesh=_mesh,
    compiler_params=_cparams,
    scratch_types=[
        pltpu.VMEM((CHUNK_E,), jnp.float32),
        pltpu.VMEM((CHUNK_E,), jnp.float32),
        pltpu.VMEM((LANES,), jnp.int32),
        pltpu.VMEM((NW * D,), jnp.float32),
        pltpu.VMEM((NW * LANES,), jnp.float32),
        pltpu.VMEM((CHUNK_B,), jnp.float32),
    ],
)


@jax.jit
def kernel(context, cu_seqlens, context_theta):
    ctx_flat = context.reshape(-1)
    th_flat = context_theta.reshape(-1)
    cu_pad = jnp.concatenate(
        [cu_seqlens.astype(jnp.int32),
         jnp.zeros((LANES - cu_seqlens.shape[0],), jnp.int32)])
    tnum, tden, ebc = _phase1(ctx_flat, cu_pad, th_flat)
    out_flat = _phase2(ctx_flat, cu_pad, tnum, tden, ebc)
    return out_flat.reshape(T, D)


# speedup vs baseline: 1.0435x; 1.0435x over previous
import jax
import jax.numpy as jnp
from jax import lax
from jax.experimental import pallas as pl
from jax.experimental.pallas import tpu as pltpu, tpu_sc as plsc

T = 4096
D = 128
LANES = 16
NS = 16
CHUNK = T // NS
CHUNK_E = CHUNK * D
CHUNK_B = CHUNK * LANES
KD = D // LANES
GROUP = 16
TAIL_W = D + LANES

_mesh = plsc.VectorSubcoreMesh(core_axis_name="c", subcore_axis_name="s",
                               num_cores=1)
_cparams = pltpu.CompilerParams(needs_layout_passes=False)


def _cu_scalars(cu_vec):
    lane = lax.iota(jnp.int32, LANES)
    cu_f = cu_vec.astype(jnp.float32)
    c1 = jnp.sum(jnp.where(lane == 1, cu_f, 0.0)).astype(jnp.int32)
    c2 = jnp.sum(jnp.where(lane == 2, cu_f, 0.0)).astype(jnp.int32)
    c3 = jnp.sum(jnp.where(lane == 3, cu_f, 0.0)).astype(jnp.int32)
    return c1, c2, c3


def _seg_of(p, c1, c2, c3):
    return ((p >= c1).astype(jnp.int32) + (p >= c2).astype(jnp.int32)
            + (p >= c3).astype(jnp.int32))


def _body(ctx_hbm, cu_hbm, th_hbm, out_hbm, tails_hbm,
          ctx_v, num_v, den_v, th_v, cu_v, tl_v, ta_v):
    w = lax.axis_index("s")
    pltpu.sync_copy(ctx_hbm.at[pl.ds(w * CHUNK_E, CHUNK_E)], ctx_v)
    pltpu.sync_copy(th_hbm, th_v)
    pltpu.sync_copy(cu_hbm, cu_v)
    c1, c2, c3 = _cu_scalars(cu_v[:])
    th = [th_v[pl.ds(LANES * k, LANES)] for k in range(KD)]
    row0 = w * CHUNK
    zero = jnp.zeros((LANES,), jnp.float32)

    def p1(g, carry):
        den = carry[0]
        nums = carry[1:]
        base = g * (GROUP * D)
        for j in range(GROUP):
            off = base + j * D
            cks = [ctx_v[pl.ds(off + LANES * k, LANES)] for k in range(KD)]
            acc = cks[0] * th[0]
            for k in range(1, KD):
                acc = acc + cks[k] * th[k]
            e = jnp.exp(jnp.full((LANES,), jnp.sum(acc), jnp.float32))
            den = den + e
            nums = tuple(n + e * ck for n, ck in zip(nums, cks))
            for k in range(KD):
                num_v[pl.ds(off + LANES * k, LANES)] = nums[k]
            den_v[pl.ds((base // KD) + j * LANES, LANES)] = den
        return (den,) + nums

    lax.fori_loop(0, CHUNK // GROUP, p1, (zero,) * (KD + 1))

    def _cut(cb):
        rel = cb - row0
        inside = (rel >= 0) & (rel < CHUNK)
        return jnp.where(inside, rel, 0)

    maxcut = jnp.maximum(_cut(c1), jnp.maximum(_cut(c2), _cut(c3)))
    mc_off = jnp.maximum(maxcut, 1) - 1
    have = jnp.where(maxcut > 0, 1.0, 0.0)
    hv = jnp.full((LANES,), have, jnp.float32)
    last = CHUNK_E - D
    for k in range(KD):
        bl = num_v[pl.ds(mc_off * D + LANES * k, LANES)]
        tl_v[pl.ds(LANES * k, LANES)] = (
            num_v[pl.ds(last + LANES * k, LANES)] - hv * bl)
    bld = den_v[pl.ds(mc_off * LANES, LANES)]
    tl_v[pl.ds(D, LANES)] = den_v[pl.ds(CHUNK_B - LANES, LANES)] - hv * bld
    pltpu.sync_copy(tl_v, tails_hbm.at[pl.ds(w * TAIL_W, TAIL_W)])
    plsc.subcore_barrier()
    pltpu.sync_copy(tails_hbm, ta_v)

    s0 = _seg_of(row0, c1, c2, c3)
    cden = zero
    cnum = [zero] * KD
    for wp in range(NS - 1):
        segl = _seg_of(wp * CHUNK + CHUNK - 1, c1, c2, c3)
        take = (wp < w) & (segl == s0)
        mv = jnp.full((LANES,), jnp.where(take, 1.0, 0.0), jnp.float32)
        for k in range(KD):
            cnum[k] = cnum[k] + mv * ta_v[pl.ds(wp * TAIL_W + LANES * k, LANES)]
        cden = cden + mv * ta_v[pl.ds(wp * TAIL_W + D, LANES)]

    def p2(g, carry):
        bden = carry[0]
        pden = carry[1]
        bnum = list(carry[2:2 + KD])
        pnum = list(carry[2 + KD:])
        base = g * (GROUP * D)
        for j in range(GROUP):
            off = base + j * D
            rg = g * GROUP + j
            is_start = (rg == (c1 - row0)) | (rg == (c2 - row0)) | \
                       (rg == (c3 - row0))
            nums = [num_v[pl.ds(off + LANES * k, LANES)] for k in range(KD)]
            den = den_v[pl.ds((base // KD) + j * LANES, LANES)]
            bden = jnp.where(is_start, pden, bden)
            for k in range(KD):
                bnum[k] = jnp.where(is_start, pnum[k], bnum[k])
            inv = 1.0 / (den - bden)
            for k in range(KD):
                num_v[pl.ds(off + LANES * k, LANES)] = (nums[k] - bnum[k]) * inv
            pden = den
            pnum = nums
        return (bden, pden) + tuple(bnum) + tuple(pnum)

    init = (-cden, zero) + tuple(-n for n in cnum) + (zero,) * KD
    lax.fori_loop(0, CHUNK // GROUP, p2, init)

    pltpu.sync_copy(num_v, out_hbm.at[pl.ds(w * CHUNK_E, CHUNK_E)])


_run = pl.kernel(
    _body,
    out_type=(jax.ShapeDtypeStruct((T * D,), jnp.float32),
              jax.ShapeDtypeStruct((NS * TAIL_W,), jnp.float32)),
    mesh=_mesh,
    compiler_params=_cparams,
    scratch_types=[
        pltpu.VMEM((CHUNK_E,), jnp.float32),
        pltpu.VMEM((CHUNK_E,), jnp.float32),
        pltpu.VMEM((CHUNK_B,), jnp.float32),
        pltpu.VMEM((D,), jnp.float32),
        pltpu.VMEM((LANES,), jnp.int32),
        pltpu.VMEM((TAIL_W,), jnp.float32),
        pltpu.VMEM((NS * TAIL_W,), jnp.float32),
    ],
)


@jax.jit
def kernel(context, cu_seqlens, context_theta):
    ctx_flat = context.reshape(-1)
    th_flat = context_theta.reshape(-1)
    cu_pad = jnp.concatenate(
        [cu_seqlens.astype(jnp.int32),
         jnp.zeros((LANES - cu_seqlens.shape[0],), jnp.int32)])
    out_flat, _ = _run(ctx_flat, cu_pad, th_flat)
    return out_flat.reshape(T, D)
